# Initial kernel scaffold; baseline (speedup 1.0000x reference)
import jax
import jax.numpy as jnp
from jax.experimental import pallas as pl


def kernel(x, edge_index, batch, Wp, bp, gW0, gb0, gg0, gbe0, gW1, gb1, gg1, gbe1, sWl0, sbl0, sWr0, sg0, sbe0, sWl1, sbl1, sWr1, sg1, sbe1, f1W, f1b, bn1g, bn1b, f2W, f2b, bn2g, bn2b, f3W, f3b):
    raise NotImplementedError("write your pallas kernel here")


# speedup vs baseline: 3.3638x; 3.3638x over previous
import functools
import jax
import jax.numpy as jnp
from jax import lax
from jax.experimental import pallas as pl
from jax.experimental.pallas import tpu as pltpu
from jax.experimental.pallas import tpu_sc as plsc

_N = 10000
_E = 320000
_F = 128
_G = 64
_EPS = 1e-5
_NS = 16
_K = 128
_EP = 327680
_CH = _EP // _NS // _K
_ZR = 64
_AH = 5120
_HN = 5000
_RT = _N // _NS
_B = 1000
_GRID = _N // _B

_f32 = jnp.float32
_MESH = plsc.VectorSubcoreMesh(core_axis_name="c", subcore_axis_name="s")


def _dot(a, b):
    def split(v):
        c = v * 65537.0
        hi = c - (c - v)
        return hi, v - hi

    ah, al = split(a)
    bh, bl = split(b)
    mm = lambda p, q: jnp.dot(p, q, preferred_element_type=_f32)
    return mm(ah, bh) + (mm(ah, bl) + mm(al, bh))


@functools.partial(
    pl.kernel,
    out_type=jax.ShapeDtypeStruct((_N, _F), _f32),
    mesh=_MESH,
    scratch_types=[
        pltpu.VMEM((_CH, _K), jnp.int32),
        pltpu.VMEM((_CH, _K), jnp.int32),
        pltpu.VMEM((_K, _F), _f32),
        pltpu.VMEM_SHARED((5120, _F), _f32),
    ],
)
def _sc_deg(dlo_hbm, dhi_hbm, ones_hbm, zrow_hbm, deg_out,
            dlo_i, dhi_i, ones_v, accd):
    cid = lax.axis_index("c")
    sid = lax.axis_index("s")

    @pl.when(cid == 0)
    def _():
        pltpu.sync_copy(dlo_hbm.at[sid], dlo_i)
        pltpu.sync_copy(dhi_hbm.at[sid], dhi_i)
        pltpu.sync_copy(ones_hbm, ones_v)

        def half(dst_i, out_base):
            @pl.when(sid < 10)
            def _():
                def zbody(k, carry):
                    pltpu.sync_copy(zrow_hbm,
                                    accd.at[pl.ds(sid * 512 + k * _ZR, _ZR)])
                    return carry

                lax.fori_loop(0, 8, zbody, 0)

            plsc.subcore_barrier()

            def body(j, carry):
                pltpu.sync_copy(ones_v, accd.at[dst_i.at[j]], add=True)
                return carry

            lax.fori_loop(0, _CH, body, 0)
            plsc.subcore_barrier()

            @pl.when(sid < 5)
            def _():
                pltpu.sync_copy(
                    accd.at[pl.ds(sid * 1000, 1000)],
                    deg_out.at[pl.ds(out_base + sid * 1000, 1000)])

            plsc.subcore_barrier()

        half(dlo_i, 0)
        half(dhi_i, _HN)


@functools.partial(
    pl.kernel,
    out_type=[jax.ShapeDtypeStruct((_N, _F), _f32),
              jax.ShapeDtypeStruct((_N, _F), _f32)],
    mesh=_MESH,
    scratch_types=[
        pltpu.VMEM((_CH, _K), jnp.int32),
        pltpu.VMEM((_CH, _K), jnp.int32),
        pltpu.VMEM((_CH, _K), jnp.int32),
        pltpu.VMEM((_K, _F), _f32),
        pltpu.VMEM_SHARED((_AH, _F), _f32),
        pltpu.SemaphoreType.DMA,
    ],
)
def _sc_scatter(ta_hbm, tb_hbm, src_hbm, dlo_hbm, dhi_hbm, zrow_hbm,
                outa, outb, src_i, dlo_i, dhi_i, rows, acc, sem):
    cid = lax.axis_index("c")
    sid = lax.axis_index("s")
    pltpu.sync_copy(src_hbm.at[sid], src_i)
    pltpu.sync_copy(dlo_hbm.at[sid], dlo_i)
    pltpu.sync_copy(dhi_hbm.at[sid], dhi_i)

    def run(table, out):
        def half(dst_i, out_base):
            @pl.when(sid < 10)
            def _():
                def zbody(k, carry):
                    pltpu.sync_copy(zrow_hbm,
                                    acc.at[pl.ds(sid * 512 + k * _ZR, _ZR)])
                    return carry

                lax.fori_loop(0, 8, zbody, 0)

            plsc.subcore_barrier()

            def body(j, carry):
                pltpu.async_copy(table.at[src_i.at[j]], rows, sem).wait()
                pltpu.sync_copy(rows, acc.at[dst_i.at[j]], add=True)
                return carry

            lax.fori_loop(0, _CH, body, 0)
            plsc.subcore_barrier()

            @pl.when(sid < 5)
            def _():
                pltpu.sync_copy(
                    acc.at[pl.ds(sid * 1000, 1000)],
                    out.at[pl.ds(out_base + sid * 1000, 1000)])

            plsc.subcore_barrier()

        half(dlo_i, 0)
        half(dhi_i, _HN)

    @pl.when(cid == 0)
    def _():
        run(ta_hbm, outa)

    @pl.when(cid == 1)
    def _():
        run(tb_hbm, outb)


_row = lambda: pl.BlockSpec((_B, _F), lambda i: (i, 0))
_row16 = lambda: pl.BlockSpec((_B, 16), lambda i: (i, 0))
_wgt = lambda: pl.BlockSpec((_F, _F), lambda i: (0, 0))
_vec = lambda: pl.BlockSpec((1, _F), lambda i: (0, 0))


def _k1_body(x_ref, Wp_ref, bp_ref, gW0_ref, h_ref, u_ref):
    h = jnp.maximum(_dot(x_ref[...], Wp_ref[...]) + bp_ref[...], 0.0)
    h_ref[...] = h
    u_ref[...] = _dot(h, gW0_ref[...])


_k1 = pl.pallas_call(
    _k1_body,
    grid=(_GRID,),
    in_specs=[pl.BlockSpec((_B, _F), lambda i: (i, 0)), _wgt(), _vec(), _wgt()],
    out_specs=[_row(), _row()],
    out_shape=[jax.ShapeDtypeStruct((_N, _F), _f32),
               jax.ShapeDtypeStruct((_N, _F), _f32)],
)


def _k2_body(deg_ref, u_ref, t0_ref, dinv_ref, sinv_ref):
    d = deg_ref[...][:, :16]
    dinv = 1.0 / jnp.sqrt(d + 1.0)
    dinv_ref[...] = dinv
    sinv_ref[...] = 1.0 / jnp.maximum(d, 1.0)
    t0_ref[...] = u_ref[...] * dinv[:, :1]


_k2 = pl.pallas_call(
    _k2_body,
    grid=(_GRID,),
    in_specs=[_row(), _row()],
    out_specs=[_row(), _row16(), _row16()],
    out_shape=[jax.ShapeDtypeStruct((_N, _F), _f32),
               jax.ShapeDtypeStruct((_N, 16), _f32),
               jax.ShapeDtypeStruct((_N, 16), _f32)],
)


def _k3_body(aggg, t, dinv, gb, aggs, sinv, xprev, Wl, Wr, bl,
             yg_ref, ys_ref, st_ref):
    i = pl.program_id(0)
    yg = (aggg[...] + t[...]) * dinv[...][:, :1] + gb[...]
    mean_agg = aggs[...] * sinv[...][:, :1]
    ys = _dot(mean_agg, Wl[...]) + _dot(xprev[...], Wr[...]) + bl[...]
    yg_ref[...] = yg
    ys_ref[...] = ys
    blk = jnp.concatenate(
        [jnp.sum(yg, 0, keepdims=True), jnp.sum(yg * yg, 0, keepdims=True),
         jnp.sum(ys, 0, keepdims=True), jnp.sum(ys * ys, 0, keepdims=True)],
        axis=0)

    @pl.when(i == 0)
    def _():
        st_ref[...] = blk

    @pl.when(i > 0)
    def _():
        st_ref[...] += blk


_k3 = pl.pallas_call(
    _k3_body,
    grid=(_GRID,),
    in_specs=[_row(), _row(), _row16(), _vec(), _row(), _row16(), _row(),
              _wgt(), _wgt(), _vec()],
    out_specs=[_row(), _row(), pl.BlockSpec((4, _F), lambda i: (0, 0))],
    out_shape=[jax.ShapeDtypeStruct((_N, _F), _f32),
               jax.ShapeDtypeStruct((_N, _F), _f32),
               jax.ShapeDtypeStruct((4, _F), _f32)],
)


def _bn_from_stats(y, s0, s1, g, b):
    m = s0 * (1.0 / _N)
    v = s1 * (1.0 / _N) - m * m
    return jnp.maximum((y - m) / jnp.sqrt(v + _EPS) * g + b, 0.0)


def _k4_body(yg, ys, st, gg, gbe, sg, sbe, gW1, dinv, t1_ref, xs1_ref):
    s = st[...]
    xg1 = _bn_from_stats(yg[...], s[0:1], s[1:2], gg[...], gbe[...])
    t1_ref[...] = _dot(xg1, gW1[...]) * dinv[...][:, :1]
    xs1_ref[...] = _bn_from_stats(ys[...], s[2:3], s[3:4], sg[...], sbe[...])


_k4 = pl.pallas_call(
    _k4_body,
    grid=(_GRID,),
    in_specs=[_row(), _row(), pl.BlockSpec((4, _F), lambda i: (0, 0)),
              _vec(), _vec(), _vec(), _vec(), _wgt(), _row16()],
    out_specs=[_row(), _row()],
    out_shape=[jax.ShapeDtypeStruct((_N, _F), _f32),
               jax.ShapeDtypeStruct((_N, _F), _f32)],
)


def _k6_body(yg, ys, st, gg, gbe, sg, sbe, batch_ref,
             pg_ref, ps_ref, cnt_ref):
    i = pl.program_id(0)
    s = st[...]
    xg = _bn_from_stats(yg[...], s[0:1], s[1:2], gg[...], gbe[...])
    xs = _bn_from_stats(ys[...], s[2:3], s[3:4], sg[...], sbe[...])
    b = batch_ref[0]
    gids = lax.broadcasted_iota(jnp.int32, (_G, _B), 0)
    onehot = (gids == b).astype(_f32)
    pgb = _dot(onehot, xg)
    psb = _dot(onehot, xs)
    cntb = _dot(onehot, jnp.ones((_B, _F), _f32))

    @pl.when(i == 0)
    def _():
        pg_ref[...] = pgb
        ps_ref[...] = psb
        cnt_ref[...] = cntb

    @pl.when(i > 0)
    def _():
        pg_ref[...] += pgb
        ps_ref[...] += psb
        cnt_ref[...] += cntb


_pool_spec = pl.BlockSpec((_G, _F), lambda i: (0, 0))
_k6 = pl.pallas_call(
    _k6_body,
    grid=(_GRID,),
    in_specs=[_row(), _row(), pl.BlockSpec((4, _F), lambda i: (0, 0)),
              _vec(), _vec(), _vec(), _vec(),
              pl.BlockSpec((1, 1, _B), lambda i: (i, 0, 0))],
    out_specs=[_pool_spec, _pool_spec, _pool_spec],
    out_shape=[jax.ShapeDtypeStruct((_G, _F), _f32),
               jax.ShapeDtypeStruct((_G, _F), _f32),
               jax.ShapeDtypeStruct((_G, _F), _f32)],
)


def _bn_full(y, g, b):
    m = jnp.mean(y, axis=0, keepdims=True)
    v = jnp.mean(y * y, axis=0, keepdims=True) - m * m
    return (y - m) / jnp.sqrt(v + _EPS) * g + b


def _k7_body(pg, ps, cnt, f1W, f1b, bn1g, bn1b, f2W, f2b, bn2g, bn2b,
             f3Wp, f3bp, out_ref):
    c = jnp.maximum(cnt[...], 1.0)
    z = jnp.concatenate([pg[...] / c, ps[...] / c], axis=1)
    a = _dot(z, f1W[...]) + f1b[...]
    a = jnp.maximum(_bn_full(a, bn1g[...], bn1b[...]), 0.0)
    b2 = _dot(a, f2W[...]) + f2b[...]
    b2 = jnp.maximum(_bn_full(b2, bn2g[...], bn2b[...]), 0.0)
    out_ref[...] = _dot(b2, f3Wp[...]) + f3bp[...]


_k7 = pl.pallas_call(
    _k7_body,
    out_shape=jax.ShapeDtypeStruct((_G, _F), _f32),
)


def kernel(x, edge_index, batch, Wp, bp, gW0, gb0, gg0, gbe0, gW1, gb1,
           gg1, gbe1, sWl0, sbl0, sWr0, sg0, sbe0, sWl1, sbl1, sWr1, sg1,
           sbe1, f1W, f1b, bn1g, bn1b, f2W, f2b, bn2g, bn2b, f3W, f3b):
    pad = _EP - _E
    src = jnp.concatenate(
        [edge_index[0], jnp.zeros((pad,), jnp.int32)]).reshape(_NS, _CH, _K)
    dst_flat = edge_index[1]
    jspread = _HN + (jnp.arange(_E, dtype=jnp.int32) % 120)
    junk = _HN + (jnp.arange(pad, dtype=jnp.int32) % 120)
    dlo = jnp.concatenate(
        [jnp.where(dst_flat < _HN, dst_flat, jspread), junk]).reshape(
        _NS, _CH, _K)
    dhi = jnp.concatenate(
        [jnp.where(dst_flat >= _HN, dst_flat - _HN, jspread), junk]).reshape(
        _NS, _CH, _K)
    ones128 = jnp.ones((_K, _F), _f32)
    zrow = jnp.zeros((_ZR, _F), _f32)
    batch3d = batch.reshape(_GRID, 1, _B)
    row1 = lambda v: v.reshape(1, -1)

    deg = _sc_deg(dlo, dhi, ones128, zrow)
    h, u = _k1(x, Wp, row1(bp), gW0)
    t0, dinv, sinv = _k2(deg, u)
    aggg1, aggs1 = _sc_scatter(t0, h, src, dlo, dhi, zrow)
    yg0, ys0, st0 = _k3(aggg1, t0, dinv, row1(gb0), aggs1, sinv, h,
                        sWl0, sWr0, row1(sbl0))
    t1, xs1 = _k4(yg0, ys0, st0, row1(gg0), row1(gbe0), row1(sg0),
                  row1(sbe0), gW1, dinv)
    aggg2, aggs2 = _sc_scatter(t1, xs1, src, dlo, dhi, zrow)
    yg1, ys1, st1 = _k3(aggg2, t1, dinv, row1(gb1), aggs2, sinv, xs1,
                        sWl1, sWr1, row1(sbl1))
    pg, ps, cnt = _k6(yg1, ys1, st1, row1(gg1), row1(gbe1), row1(sg1),
                      row1(sbe1), batch3d)
    f3Wp = jnp.pad(f3W, ((0, 0), (0, _F - 1)))
    f3bp = jnp.pad(f3b, (0, _F - 1))
    out = _k7(pg, ps, cnt, f1W, row1(f1b), row1(bn1g), row1(bn1b),
              f2W, row1(f2b), row1(bn2g), row1(bn2b), f3Wp, row1(f3bp))
    return out[:, 0]
